# baseline (speedup 1.0000x reference)
import functools

import jax
import jax.numpy as jnp
from jax import lax
from jax.experimental import pallas as pl
from jax.experimental.pallas import tpu as pltpu
from jax.experimental.pallas import tpu_sc as plsc

_NC = 2
_NS = 16
_L = 16
_NB = 256
_UNROLL = 8
_CCAP = 16384


def _i32(x):
    return x.astype(jnp.int32)


def _splat(x, dtype=jnp.int32):
    return jnp.full((_L,), x, dtype=dtype)


def _merge_level(hist_v, mbuf_v, need):
    zero_i = _splat(0)
    lane = lax.iota(jnp.int32, _L)

    def mg_body(g, _):
        acc = zero_i
        for l in range(_NS):
            sl = pl.ds(l * _NB + g * _L, _L)
            acc = acc + hist_v[sl]
            hist_v[sl] = zero_i
        mbuf_v[pl.ds(g * _L, _L)] = acc
        return 0

    lax.fori_loop(0, _NB // _L, mg_body, 0)

    need_s = _splat(need)

    def pick_body(t, carry):
        run, acc_cnt = carry
        g = (_NB // _L - 1) - t
        v = mbuf_v[pl.ds(g * _L, _L)]
        wg = lax.rev(plsc.cumsum(lax.rev(v, (0,))), (0,))
        suffix = wg + _splat(run)
        acc_cnt = acc_cnt + _i32(suffix >= need_s)
        return run + jnp.sum(v), acc_cnt

    _, acc_cnt = lax.fori_loop(0, _NB // _L, pick_body,
                               (jnp.int32(0), zero_i))
    d_star = jnp.sum(acc_cnt) - 1
    ds = _splat(d_star)

    def cnt_body(g, carry):
        acc_above, acc_at = carry
        idx_g = lane + _splat(g * _L)
        v = mbuf_v[pl.ds(g * _L, _L)]
        acc_above = acc_above + jnp.where(idx_g > ds, v, zero_i)
        acc_at = acc_at + jnp.where(idx_g == ds, v, zero_i)
        return acc_above, acc_at

    acc_above, acc_at = lax.fori_loop(0, _NB // _L, cnt_body, (zero_i, zero_i))
    return d_star, jnp.sum(acc_above), jnp.sum(acc_at)


def _make_sc_kernel(b, n, k):
    rows_per_w = b // (_NC * _NS)
    n_chunks = n // (_L * _UNROLL)
    mesh = plsc.VectorSubcoreMesh(core_axis_name="c", subcore_axis_name="s")

    @functools.partial(
        pl.kernel,
        out_type=jax.ShapeDtypeStruct((b, n), jnp.float32),
        mesh=mesh,
        scratch_types=[
            pltpu.VMEM((n,), jnp.float32),
            pltpu.VMEM((_CCAP + 4 * _L,), jnp.int32),
            pltpu.VMEM((_NS * _NB,), jnp.int32),
            pltpu.VMEM((_NB,), jnp.int32),
        ],
        compiler_params=pltpu.CompilerParams(needs_layout_passes=False),
    )
    def sc_kernel(x_hbm, out_hbm, row_v, cbuf_v, hist_v, mbuf_v):
        wid = lax.axis_index("s") * _NC + lax.axis_index("c")
        lane = lax.iota(jnp.int32, _L)
        lane_off = lane * _NB
        ones_i = _splat(1)
        zero_i = _splat(0)
        mask7f = jnp.int32(0x7FFFFFFF)

        def zero_body(i, _):
            hist_v[pl.ds(i * _L, _L)] = zero_i
            return 0

        lax.fori_loop(0, _NS * _NB // _L, zero_body, 0)

        def row_body(r, _):
            row = wid * rows_per_w + r
            pltpu.sync_copy(x_hbm.at[row], row_v)

            def l0_body(i, _):
                base = i * (_L * _UNROLL)
                for u in range(_UNROLL):
                    sl = pl.ds(base + u * _L, _L)
                    bits = plsc.bitcast(row_v[sl], jnp.int32)
                    key = bits ^ ((bits >> 31) & mask7f)
                    row_v[sl] = plsc.bitcast(key, jnp.float32)
                    d = (key >> 24) + _splat(128)
                    plsc.addupdate_scatter(hist_v, [lane_off + d], ones_i)
                return 0

            lax.fori_loop(0, n_chunks, l0_body, 0)
            d0, above0, at0 = _merge_level(hist_v, mbuf_v, jnp.int32(k))
            pref8 = d0 - 128
            a1 = above0
            m1 = at0

            p8 = _splat(pref8)
            ccap = _splat(_CCAP)

            def l1_body(i, off):
                base = i * (_L * _UNROLL)
                for u in range(_UNROLL):
                    sl = pl.ds(base + u * _L, _L)
                    kv = plsc.bitcast(row_v[sl], jnp.int32)
                    m = (kv >> 24) == p8
                    d = (kv >> 16) & 255
                    plsc.addupdate_scatter(hist_v, [lane_off + d], ones_i,
                                           mask=m)
                    m01 = _i32(m)
                    pos = off + plsc.cumsum(m01) - m01
                    pos = jnp.minimum(pos, ccap)
                    plsc.store_scatter(cbuf_v, [pos], kv, mask=m)
                    off = off + plsc.all_reduce_population_count(m)
                return off

            lax.fori_loop(0, n_chunks, l1_body, zero_i)
            d1, above1, at1 = _merge_level(hist_v, mbuf_v, jnp.int32(k) - a1)
            pref16 = (pref8 << 8) | d1
            a2 = a1 + above1
            m2 = at1

            compacted = m1 <= jnp.int32(_CCAP)

            @pl.when(compacted)
            def _():
                p16 = _splat(pref16)
                m1s = _splat(m1)

                def body(i, _):
                    base = i * (_L * 4)
                    for u in range(4):
                        pos0 = base + u * _L
                        kv = cbuf_v[pl.ds(pos0, _L)]
                        valid = (lane + _splat(pos0)) < m1s
                        m = valid & ((kv >> 16) == p16)
                        d = (kv >> 8) & 255
                        plsc.addupdate_scatter(hist_v, [lane_off + d], ones_i,
                                               mask=m)
                    return 0

                lax.fori_loop(0, (m1 + _L * 4 - 1) // (_L * 4), body, 0)

            @pl.when(jnp.logical_not(compacted))
            def _():
                p16 = _splat(pref16)

                def body(i, _):
                    base = i * (_L * _UNROLL)
                    for u in range(_UNROLL):
                        kv = plsc.bitcast(row_v[pl.ds(base + u * _L, _L)],
                                          jnp.int32)
                        m = (kv >> 16) == p16
                        d = (kv >> 8) & 255
                        plsc.addupdate_scatter(hist_v, [lane_off + d], ones_i,
                                               mask=m)
                    return 0

                lax.fori_loop(0, n_chunks, body, 0)

            d2, above2, at2 = _merge_level(hist_v, mbuf_v, jnp.int32(k) - a2)
            pref24 = (pref16 << 8) | d2
            a3 = a2 + above2

            @pl.when(compacted)
            def _():
                p24 = _splat(pref24)
                m1s = _splat(m1)

                def body(i, _):
                    base = i * (_L * 4)
                    for u in range(4):
                        pos0 = base + u * _L
                        kv = cbuf_v[pl.ds(pos0, _L)]
                        valid = (lane + _splat(pos0)) < m1s
                        m = valid & ((kv >> 8) == p24)
                        d = kv & 255
                        plsc.addupdate_scatter(hist_v, [lane_off + d], ones_i,
                                               mask=m)
                    return 0

                lax.fori_loop(0, (m1 + _L * 4 - 1) // (_L * 4), body, 0)

            @pl.when(jnp.logical_not(compacted))
            def _():
                p24 = _splat(pref24)

                def body(i, _):
                    base = i * (_L * _UNROLL)
                    for u in range(_UNROLL):
                        kv = plsc.bitcast(row_v[pl.ds(base + u * _L, _L)],
                                          jnp.int32)
                        m = (kv >> 8) == p24
                        d = kv & 255
                        plsc.addupdate_scatter(hist_v, [lane_off + d], ones_i,
                                               mask=m)
                    return 0

                lax.fori_loop(0, n_chunks, body, 0)

            d3, _, _ = _merge_level(hist_v, mbuf_v, jnp.int32(k) - a3)
            thresh = _splat((pref24 << 8) | d3)

            one_f = _splat(1.0, jnp.float32)
            zero_f = _splat(0.0, jnp.float32)

            def mask_body(i, _):
                base = i * (_L * _UNROLL)
                for u in range(_UNROLL):
                    sl = pl.ds(base + u * _L, _L)
                    kv = plsc.bitcast(row_v[sl], jnp.int32)
                    row_v[sl] = jnp.where(kv >= thresh, one_f, zero_f)
                return 0

            lax.fori_loop(0, n_chunks, mask_body, 0)
            pltpu.sync_copy(row_v, out_hbm.at[row])
            return 0

        lax.fori_loop(0, rows_per_w, row_body, 0)

    return sc_kernel


@jax.jit
def kernel(attention_scores):
    b, n = attention_scores.shape
    k = max(1, int(n * 0.3))
    return _make_sc_kernel(b, n, k)(attention_scores)
